# baseline (speedup 1.0000x reference)
import jax
import jax.numpy as jnp
from jax.experimental import pallas as pl

B, X, C_IN = 8, 16, 32
K, S, F = 3, 2, 64
OX = (X - K) // S + 1
NPOS = OX * OX * OX
CK = K * K * K * C_IN
PB = 7


def _im2col(x):
    slices = []
    for i in range(K):
        for j in range(K):
            for l in range(K):
                slices.append(x[:, i:i + S * (OX - 1) + 1:S,
                                  j:j + S * (OX - 1) + 1:S,
                                  l:l + S * (OX - 1) + 1:S, :])
    p = jnp.stack(slices, axis=0)
    p = p.transpose(2, 3, 4, 1, 0, 5)
    return p.reshape(NPOS, B, CK)


def _body(p_ref, ps_ref, loc_ref, rho_ref, eps_ref, b_ref, so_ref, o_ref):
    for j in range(PB):
        w2 = jax.nn.softplus(rho_ref[j]) * eps_ref[j]
        m = jnp.dot(p_ref[j], loc_ref[j], preferred_element_type=jnp.float32)
        pert = jnp.dot(ps_ref[j], w2, preferred_element_type=jnp.float32)
        o_ref[j] = m + b_ref[j] + pert * so_ref[j]


def kernel(inputs, kernel_loc, kernel_rho, bias, eps, sign_in, sign_out):
    patches = _im2col(inputs)
    patches_s = _im2col(inputs * sign_in)
    loc = kernel_loc.reshape(NPOS, CK, F)
    rho = kernel_rho.reshape(NPOS, CK, F)
    epsr = eps.reshape(NPOS, CK, F)
    bias3 = bias.reshape(NPOS, 1, F)
    so = sign_out.transpose(1, 2, 3, 0, 4).reshape(NPOS, B, F)

    grid = (NPOS // PB,)
    wspec = pl.BlockSpec((PB, CK, F), lambda i: (i, 0, 0))
    pspec = pl.BlockSpec((PB, B, CK), lambda i: (i, 0, 0))
    out = pl.pallas_call(
        _body,
        grid=grid,
        in_specs=[
            pspec, pspec, wspec, wspec, wspec,
            pl.BlockSpec((PB, 1, F), lambda i: (i, 0, 0)),
            pl.BlockSpec((PB, B, F), lambda i: (i, 0, 0)),
        ],
        out_specs=pl.BlockSpec((PB, B, F), lambda i: (i, 0, 0)),
        out_shape=jax.ShapeDtypeStruct((NPOS, B, F), jnp.float32),
    )(patches, patches_s, loc, rho, epsr, bias3, so)

    return out.reshape(OX, OX, OX, B, F).transpose(3, 0, 1, 2, 4)
